# baseline (speedup 1.0000x reference)
import functools

import jax
import jax.numpy as jnp
from jax import lax
from jax.experimental import pallas as pl
from jax.experimental.pallas import tpu as pltpu
from jax.experimental.pallas import tpu_sc as plsc

_B, _C, _N, _K = 1, 128, 10000, 32
_OUT = 128
_NW = 32
_NPAD = 10240
_CH = 4
_EPC = _CH * _K
_BLANES = 32
_D = _OUT // _BLANES

_PAIR_CHUNKS = _NPAD // 16 // _CH
_CF = 104
_CS = _PAIR_CHUNKS - _CF

_NBUF = 3
_NOBUF = 2


def _tables_body(x_ref, w_ref, b_ref, u_ref, v_ref):
    w1 = w_ref[:, :_C]
    w2 = w_ref[:, _C:]
    a = w1 - w2
    xb = x_ref[...]
    dn = (((0,), (1,)), ((), ()))
    u = lax.dot_general(xb, a, dn, precision=lax.Precision.HIGHEST,
                        preferred_element_type=jnp.float32)
    v = lax.dot_general(xb, w2, dn, precision=lax.Precision.HIGHEST,
                        preferred_element_type=jnp.float32)
    u_ref[...] = (u + b_ref[...]).astype(jnp.bfloat16)
    v_ref[...] = v.astype(jnp.bfloat16)


_TBLK = 2048


def _make_tables(x2, w, b2):
    grid = (_NPAD // _TBLK,)
    return pl.pallas_call(
        _tables_body,
        grid=grid,
        in_specs=[
            pl.BlockSpec((_C, _TBLK), lambda i: (0, i)),
            pl.BlockSpec((_OUT, 2 * _C), lambda i: (0, 0)),
            pl.BlockSpec((1, _OUT), lambda i: (0, 0)),
        ],
        out_specs=[
            pl.BlockSpec((_TBLK, _OUT), lambda i: (i, 0)),
            pl.BlockSpec((_TBLK, _OUT), lambda i: (i, 0)),
        ],
        out_shape=[
            jax.ShapeDtypeStruct((_NPAD, _OUT), jnp.bfloat16),
            jax.ShapeDtypeStruct((_NPAD, _OUT), jnp.bfloat16),
        ],
    )(x2, w, b2)


def _edge_body(u_hbm, v_hbm, i1_hbm, i0_hbm, out_hbm, i1w, i0w, buf, obuf,
               sem_u, sem_v, sem_s):
    c = lax.axis_index("c")
    s = lax.axis_index("s")
    my_chunks = jnp.where(c == 0, _CF, _CS)
    chunk_base = s * _PAIR_CHUNKS + c * _CF

    pltpu.sync_copy(i1_hbm.at[pl.ds(chunk_base, _CF)], i1w)
    pltpu.sync_copy(i0_hbm.at[pl.ds(chunk_base, _CF)], i0w)

    def u_desc(g):
        base = lax.rem(g, _NBUF) * _EPC
        return pltpu.make_async_copy(
            u_hbm.at[i1w.at[g]], buf.at[pl.ds(base, _EPC)], sem_u
        )

    def v_desc(g):
        base = lax.rem(g, _NBUF) * _EPC
        return pltpu.make_async_copy(
            v_hbm.at[i0w.at[g]], buf.at[pl.ds(base, _EPC)], sem_v
        )

    def s_desc(p):
        node_base = (chunk_base + 2 * p) * _CH
        obase = lax.rem(p, _NOBUF) * 2 * _CH
        return pltpu.make_async_copy(
            obuf.at[pl.ds(obase, 2 * _CH)],
            out_hbm.at[pl.ds(node_base, 2 * _CH)],
            sem_s,
        )

    u_desc(0).start()
    u_desc(0).wait()
    v_desc(0).start(add=True)
    u_desc(1).start()

    def chunk_body(g, carry):
        @pl.when(g + 1 < my_chunks)
        def _():
            u_desc(g + 1).wait()
            v_desc(g + 1).start(add=True)

        @pl.when(g + 2 < my_chunks)
        def _():
            u_desc(g + 2).start()

        v_desc(g).wait()

        p = g // 2

        @pl.when((lax.rem(g, 2) == 0) & (p >= _NOBUF))
        def _():
            s_desc(p - _NOBUF).wait()

        bufbase = lax.rem(g, _NBUF) * _EPC
        obase = lax.rem(p, _NOBUF) * 2 * _CH + lax.rem(g, 2) * _CH
        for n in range(_CH):
            base = bufbase + n * _K

            def kstep(k, accs, base=base):
                return tuple(
                    jnp.maximum(
                        accs[d], buf[base + k, pl.ds(d * _BLANES, _BLANES)]
                    )
                    for d in range(_D)
                )

            zeros = tuple(
                jnp.zeros((_BLANES,), jnp.bfloat16) for _ in range(_D)
            )
            accs = lax.fori_loop(0, _K, kstep, zeros)
            for d in range(_D):
                obuf[obase + n, pl.ds(d * _BLANES, _BLANES)] = accs[d]

        @pl.when(lax.rem(g, 2) == 1)
        def _():
            s_desc(p).start()

        return carry

    lax.fori_loop(0, my_chunks, chunk_body, 0)

    s_desc(my_chunks // 2 - 2).wait()
    s_desc(my_chunks // 2 - 1).wait()


_edge_kernel = functools.partial(
    pl.kernel,
    out_type=jax.ShapeDtypeStruct((_NPAD, _OUT), jnp.bfloat16),
    mesh=plsc.VectorSubcoreMesh(core_axis_name="c", subcore_axis_name="s"),
    compiler_params=pltpu.CompilerParams(use_tc_tiling_on_sc=False),
    scratch_types=[
        pltpu.VMEM((_CF, 128), jnp.int32),
        pltpu.VMEM((_CF, 128), jnp.int32),
        pltpu.VMEM((_NBUF * _EPC, _OUT), jnp.bfloat16),
        pltpu.VMEM((_NOBUF * 2 * _CH, _OUT), jnp.bfloat16),
        pltpu.SemaphoreType.DMA,
        pltpu.SemaphoreType.DMA,
        pltpu.SemaphoreType.DMA,
    ],
)(_edge_body)


def kernel(x, edge_index, W, b):
    x2 = x[0, :, :, 0]
    x2 = jnp.pad(x2, ((0, 0), (0, _NPAD - _N)))
    u, v = _make_tables(x2, W, b.reshape(1, _OUT))

    def prep_idx(idx):
        idx = jnp.pad(idx, ((0, _NPAD - _N), (0, 0)))
        idx = idx.reshape(_NPAD * _K // 128, 128)
        return jnp.pad(idx, ((0, _CF), (0, 0)))

    i1 = prep_idx(edge_index[1, 0])
    i0 = prep_idx(edge_index[0, 0])
    outb = _edge_kernel(u, v, i1, i0)
    out = outb[:_N].astype(jnp.float32).T
    return out[None, :, :, None]


# speedup vs baseline: 1.0697x; 1.0697x over previous
import functools

import jax
import jax.numpy as jnp
from jax import lax
from jax.experimental import pallas as pl
from jax.experimental.pallas import tpu as pltpu
from jax.experimental.pallas import tpu_sc as plsc

_B, _C, _N, _K = 1, 128, 10000, 32
_OUT = 128
_NW = 32
_NPAD = 10240
_CH = 4
_EPC = _CH * _K
_BLANES = 32
_D = _OUT // _BLANES

_PAIR_CHUNKS = _NPAD // 16 // _CH
_CF = 116
_CS = _PAIR_CHUNKS - _CF

_NBUF = 3
_NOBUF = 2


def _tables_body(x_ref, w_ref, b_ref, u_ref, v_ref):
    w1 = w_ref[:, :_C]
    w2 = w_ref[:, _C:]
    a = w1 - w2
    xb = x_ref[...]
    dn = (((0,), (1,)), ((), ()))
    u = lax.dot_general(xb, a, dn, precision=lax.Precision.HIGHEST,
                        preferred_element_type=jnp.float32)
    v = lax.dot_general(xb, w2, dn, precision=lax.Precision.HIGHEST,
                        preferred_element_type=jnp.float32)
    u_ref[...] = (u + b_ref[...]).astype(jnp.bfloat16)
    v_ref[...] = v.astype(jnp.bfloat16)


_TBLK = 2048


def _make_tables(x2, w, b2):
    grid = (_NPAD // _TBLK,)
    return pl.pallas_call(
        _tables_body,
        grid=grid,
        in_specs=[
            pl.BlockSpec((_C, _TBLK), lambda i: (0, i)),
            pl.BlockSpec((_OUT, 2 * _C), lambda i: (0, 0)),
            pl.BlockSpec((1, _OUT), lambda i: (0, 0)),
        ],
        out_specs=[
            pl.BlockSpec((_TBLK, _OUT), lambda i: (i, 0)),
            pl.BlockSpec((_TBLK, _OUT), lambda i: (i, 0)),
        ],
        out_shape=[
            jax.ShapeDtypeStruct((_NPAD, _OUT), jnp.bfloat16),
            jax.ShapeDtypeStruct((_NPAD, _OUT), jnp.bfloat16),
        ],
    )(x2, w, b2)


def _edge_body(u_hbm, v_hbm, i1_hbm, i0_hbm, out_hbm, i1w, i0w, buf, obuf,
               sem_u, sem_v, sem_s):
    c = lax.axis_index("c")
    s = lax.axis_index("s")
    my_chunks = jnp.where(c == 0, _CF, _CS)
    chunk_base = s * _PAIR_CHUNKS + c * _CF

    pltpu.sync_copy(i1_hbm.at[pl.ds(chunk_base, _CF)], i1w)
    pltpu.sync_copy(i0_hbm.at[pl.ds(chunk_base, _CF)], i0w)

    def u_desc(g):
        base = lax.rem(g, _NBUF) * _EPC
        return pltpu.make_async_copy(
            u_hbm.at[i1w.at[g]], buf.at[pl.ds(base, _EPC)], sem_u
        )

    def v_desc(g):
        base = lax.rem(g, _NBUF) * _EPC
        return pltpu.make_async_copy(
            v_hbm.at[i0w.at[g]], buf.at[pl.ds(base, _EPC)], sem_v
        )

    def s_desc(p):
        node_base = (chunk_base + 2 * p) * _CH
        obase = lax.rem(p, _NOBUF) * 2 * _CH
        return pltpu.make_async_copy(
            obuf.at[pl.ds(obase, 2 * _CH)],
            out_hbm.at[pl.ds(node_base, 2 * _CH)],
            sem_s,
        )

    u_desc(0).start()
    u_desc(0).wait()
    v_desc(0).start(add=True)
    u_desc(1).start()

    def chunk_body(g, carry):
        @pl.when(g + 1 < my_chunks)
        def _():
            u_desc(g + 1).wait()
            v_desc(g + 1).start(add=True)

        @pl.when(g + 2 < my_chunks)
        def _():
            u_desc(g + 2).start()

        v_desc(g).wait()

        p = g // 2

        @pl.when((lax.rem(g, 2) == 0) & (p >= _NOBUF))
        def _():
            s_desc(p - _NOBUF).wait()

        bufbase = lax.rem(g, _NBUF) * _EPC
        obase = lax.rem(p, _NOBUF) * 2 * _CH + lax.rem(g, 2) * _CH
        for n in range(_CH):
            base = bufbase + n * _K

            def kstep(k, accs, base=base):
                return tuple(
                    jnp.maximum(
                        accs[d], buf[base + k, pl.ds(d * _BLANES, _BLANES)]
                    )
                    for d in range(_D)
                )

            zeros = tuple(
                jnp.zeros((_BLANES,), jnp.bfloat16) for _ in range(_D)
            )
            accs = lax.fori_loop(0, _K, kstep, zeros)
            for d in range(_D):
                obuf[obase + n, pl.ds(d * _BLANES, _BLANES)] = accs[d]

        @pl.when(lax.rem(g, 2) == 1)
        def _():
            s_desc(p).start()

        return carry

    lax.fori_loop(0, my_chunks, chunk_body, 0)

    s_desc(my_chunks // 2 - 2).wait()
    s_desc(my_chunks // 2 - 1).wait()


_edge_kernel = functools.partial(
    pl.kernel,
    out_type=jax.ShapeDtypeStruct((_NPAD, _OUT), jnp.bfloat16),
    mesh=plsc.VectorSubcoreMesh(core_axis_name="c", subcore_axis_name="s"),
    compiler_params=pltpu.CompilerParams(use_tc_tiling_on_sc=False),
    scratch_types=[
        pltpu.VMEM((_CF, 128), jnp.int32),
        pltpu.VMEM((_CF, 128), jnp.int32),
        pltpu.VMEM((_NBUF * _EPC, _OUT), jnp.bfloat16),
        pltpu.VMEM((_NOBUF * 2 * _CH, _OUT), jnp.bfloat16),
        pltpu.SemaphoreType.DMA,
        pltpu.SemaphoreType.DMA,
        pltpu.SemaphoreType.DMA,
    ],
)(_edge_body)


def kernel(x, edge_index, W, b):
    x2 = x[0, :, :, 0]
    x2 = jnp.pad(x2, ((0, 0), (0, _NPAD - _N)))
    u, v = _make_tables(x2, W, b.reshape(1, _OUT))

    def prep_idx(idx):
        idx = jnp.pad(idx, ((0, _NPAD - _N), (0, 0)))
        idx = idx.reshape(_NPAD * _K // 128, 128)
        return jnp.pad(idx, ((0, _CF), (0, 0)))

    i1 = prep_idx(edge_index[1, 0])
    i0 = prep_idx(edge_index[0, 0])
    outb = _edge_kernel(u, v, i1, i0)
    out = outb[:_N].astype(jnp.float32).T
    return out[None, :, :, None]
